# baseline (speedup 1.0000x reference)
import jax
import jax.numpy as jnp
from jax.experimental import pallas as pl
from jax.experimental.pallas import tpu as pltpu

_C = 4
_H = 32
_PATCH = 36
_PACK = 32
_KIN = _PATCH * _PACK
_NH = _H * _PACK
_NOUT = _C * _PACK
_TM = 1024


def _mlp_kernel(x_ref, w1_ref, b1_ref, w2_ref, b2_ref, o_ref):
    h = jnp.dot(x_ref[...], w1_ref[...], preferred_element_type=jnp.float32)
    h = jnp.maximum(h + b1_ref[...], 0.0)
    o_ref[...] = (
        jnp.dot(h, w2_ref[...], preferred_element_type=jnp.float32) + b2_ref[...]
    )


def kernel(neighborhoods, w1_pad, w2_pad):
    n = neighborhoods.shape[0]
    flat = neighborhoods.astype(jnp.float32).reshape(n, _PATCH)
    n_pad = pl.cdiv(n, _PACK * _TM) * (_PACK * _TM)
    if n_pad != n:
        flat = jnp.pad(flat, ((0, n_pad - n), (0, 0)))
    r = n_pad // _PACK
    x = flat.reshape(r, _KIN)

    w1f = w1_pad[:_PATCH, :_H]
    b1 = w1_pad[_PATCH, :_H]
    w2f = w2_pad[:_H, :_C]
    b2 = w2_pad[_H, :_C]

    eye = jnp.eye(_PACK, dtype=jnp.float32)
    w1bd = jnp.kron(eye, w1f)
    w2bd = jnp.kron(eye, w2f)
    b1bd = jnp.tile(b1, _PACK)[None, :]
    b2bd = jnp.tile(b2, _PACK)[None, :]

    out = pl.pallas_call(
        _mlp_kernel,
        out_shape=jax.ShapeDtypeStruct((r, _NOUT), jnp.float32),
        grid=(r // _TM,),
        in_specs=[
            pl.BlockSpec((_TM, _KIN), lambda i: (i, 0)),
            pl.BlockSpec((_KIN, _NH), lambda i: (0, 0)),
            pl.BlockSpec((1, _NH), lambda i: (0, 0)),
            pl.BlockSpec((_NH, _NOUT), lambda i: (0, 0)),
            pl.BlockSpec((1, _NOUT), lambda i: (0, 0)),
        ],
        out_specs=pl.BlockSpec((_TM, _NOUT), lambda i: (i, 0)),
        compiler_params=pltpu.CompilerParams(dimension_semantics=("parallel",)),
    )(x, w1bd, b1bd, w2bd, b2bd)
    return out.reshape(n_pad, _C)[:n]


# speedup vs baseline: 46.6502x; 46.6502x over previous
import jax
import jax.numpy as jnp
from jax.experimental import pallas as pl
from jax.experimental.pallas import tpu as pltpu

_C = 4
_H = 32
_PATCH = 36
_TN = 8192


def _mlp_kernel(x_ref, w1t_ref, b1_ref, w2t_ref, b2_ref, o_ref):
    h = jnp.dot(w1t_ref[...], x_ref[...], preferred_element_type=jnp.float32)
    h = jnp.maximum(h + b1_ref[...], 0.0)
    o_ref[...] = (
        jnp.dot(w2t_ref[...], h, preferred_element_type=jnp.float32) + b2_ref[...]
    )


def kernel(neighborhoods, w1_pad, w2_pad):
    n = neighborhoods.shape[0]
    xt = jnp.transpose(neighborhoods.astype(jnp.float32), (1, 2, 3, 0))
    xt = xt.reshape(_PATCH, n)
    n_pad = pl.cdiv(n, _TN) * _TN
    if n_pad != n:
        xt = jnp.pad(xt, ((0, 0), (0, n_pad - n)))

    w1t = jnp.transpose(w1_pad[:_PATCH, :_H])
    b1 = w1_pad[_PATCH, :_H][:, None]
    w2t = jnp.transpose(w2_pad[:_H, :_C])
    b2 = w2_pad[_H, :_C][:, None]

    grid = n_pad // _TN
    out = pl.pallas_call(
        _mlp_kernel,
        out_shape=jax.ShapeDtypeStruct((_C, n_pad), jnp.float32),
        grid=(grid,),
        in_specs=[
            pl.BlockSpec((_PATCH, _TN), lambda i: (0, i)),
            pl.BlockSpec((_H, _PATCH), lambda i: (0, 0)),
            pl.BlockSpec((_H, 1), lambda i: (0, 0)),
            pl.BlockSpec((_C, _H), lambda i: (0, 0)),
            pl.BlockSpec((_C, 1), lambda i: (0, 0)),
        ],
        out_specs=pl.BlockSpec((_C, _TN), lambda i: (0, i)),
        compiler_params=pltpu.CompilerParams(dimension_semantics=("parallel",)),
    )(xt, w1t, b1, w2t, b2)
    return jnp.transpose(out[:, :n])


# speedup vs baseline: 104.8198x; 2.2469x over previous
import jax
import jax.numpy as jnp
from jax.experimental import pallas as pl
from jax.experimental.pallas import tpu as pltpu

_C = 4
_H = 32
_PATCH = 36
_TN = 8192


def _mlp_kernel(x_ref, w1t_ref, b1_ref, w2t_ref, b2_ref, o_ref):
    x = x_ref[...].reshape(_PATCH, x_ref.shape[-1])
    h = jnp.dot(w1t_ref[...], x, preferred_element_type=jnp.float32)
    h = jnp.maximum(h + b1_ref[...], 0.0)
    o_ref[...] = (
        jnp.dot(w2t_ref[...], h, preferred_element_type=jnp.float32) + b2_ref[...]
    )


def kernel(neighborhoods, w1_pad, w2_pad):
    n = neighborhoods.shape[0]
    xt = jnp.transpose(neighborhoods.astype(jnp.float32), (1, 2, 3, 0))
    xt = xt.reshape(9, 4, n)
    n_pad = pl.cdiv(n, _TN) * _TN
    if n_pad != n:
        xt = jnp.pad(xt, ((0, 0), (0, 0), (0, n_pad - n)))

    w1t = jnp.transpose(w1_pad[:_PATCH, :_H])
    b1 = w1_pad[_PATCH, :_H][:, None]
    w2t = jnp.transpose(w2_pad[:_H, :_C])
    b2 = w2_pad[_H, :_C][:, None]

    grid = n_pad // _TN
    out = pl.pallas_call(
        _mlp_kernel,
        out_shape=jax.ShapeDtypeStruct((_C, n_pad), jnp.float32),
        grid=(grid,),
        in_specs=[
            pl.BlockSpec((9, 4, _TN), lambda i: (0, 0, i)),
            pl.BlockSpec((_H, _PATCH), lambda i: (0, 0)),
            pl.BlockSpec((_H, 1), lambda i: (0, 0)),
            pl.BlockSpec((_C, _H), lambda i: (0, 0)),
            pl.BlockSpec((_C, 1), lambda i: (0, 0)),
        ],
        out_specs=pl.BlockSpec((_C, _TN), lambda i: (0, i)),
        compiler_params=pltpu.CompilerParams(dimension_semantics=("parallel",)),
    )(xt, w1t, b1, w2t, b2)
    return jnp.transpose(out[:, :n])


# speedup vs baseline: 165.1354x; 1.5754x over previous
import jax
import jax.numpy as jnp
from jax.experimental import pallas as pl
from jax.experimental.pallas import tpu as pltpu

_C = 4
_H = 32
_PATCH = 36
_TN = 32768


def _mlp_kernel(x_ref, w1t_ref, b1_ref, w2t_ref, b2_ref, o_ref):
    x = x_ref[...].reshape(_PATCH, x_ref.shape[-1])
    h = jnp.dot(w1t_ref[...], x, preferred_element_type=jnp.float32)
    h = jnp.maximum(h + b1_ref[...], 0.0)
    o_ref[...] = (
        jnp.dot(w2t_ref[...], h, preferred_element_type=jnp.float32) + b2_ref[...]
    )


def kernel(neighborhoods, w1_pad, w2_pad):
    n = neighborhoods.shape[0]
    xt = jnp.transpose(neighborhoods.astype(jnp.float32), (1, 2, 3, 0))
    xt = xt.reshape(9, 4, n)
    n_pad = pl.cdiv(n, _TN) * _TN
    if n_pad != n:
        xt = jnp.pad(xt, ((0, 0), (0, 0), (0, n_pad - n)))

    w1t = jnp.transpose(w1_pad[:_PATCH, :_H])
    b1 = w1_pad[_PATCH, :_H][:, None]
    w2t = jnp.transpose(w2_pad[:_H, :_C])
    b2 = w2_pad[_H, :_C][:, None]

    grid = n_pad // _TN
    out = pl.pallas_call(
        _mlp_kernel,
        out_shape=jax.ShapeDtypeStruct((_C, n_pad), jnp.float32),
        grid=(grid,),
        in_specs=[
            pl.BlockSpec((9, 4, _TN), lambda i: (0, 0, i)),
            pl.BlockSpec((_H, _PATCH), lambda i: (0, 0)),
            pl.BlockSpec((_H, 1), lambda i: (0, 0)),
            pl.BlockSpec((_C, _H), lambda i: (0, 0)),
            pl.BlockSpec((_C, 1), lambda i: (0, 0)),
        ],
        out_specs=pl.BlockSpec((_C, _TN), lambda i: (0, i)),
        compiler_params=pltpu.CompilerParams(dimension_semantics=("parallel",)),
    )(xt, w1t, b1, w2t, b2)
    return jnp.transpose(out[:, :n])


# speedup vs baseline: 176.0746x; 1.0662x over previous
import jax
import jax.numpy as jnp
from jax.experimental import pallas as pl
from jax.experimental.pallas import tpu as pltpu

_C = 4
_H = 32
_PATCH = 36
_TN = 65536


def _mlp_kernel(x_ref, w1t_ref, b1_ref, w2t_ref, b2_ref, o_ref):
    x = x_ref[...].reshape(_PATCH, x_ref.shape[-1])
    h = jnp.dot(w1t_ref[...], x, preferred_element_type=jnp.float32)
    h = jnp.maximum(h + b1_ref[...], 0.0)
    o_ref[...] = (
        jnp.dot(w2t_ref[...], h, preferred_element_type=jnp.float32) + b2_ref[...]
    )


def kernel(neighborhoods, w1_pad, w2_pad):
    n = neighborhoods.shape[0]
    xt = jnp.transpose(neighborhoods.astype(jnp.float32), (1, 2, 3, 0))
    xt = xt.reshape(9, 4, n)
    n_pad = pl.cdiv(n, _TN) * _TN
    if n_pad != n:
        xt = jnp.pad(xt, ((0, 0), (0, 0), (0, n_pad - n)))

    w1t = jnp.transpose(w1_pad[:_PATCH, :_H])
    b1 = w1_pad[_PATCH, :_H][:, None]
    w2t = jnp.transpose(w2_pad[:_H, :_C])
    b2 = w2_pad[_H, :_C][:, None]

    grid = n_pad // _TN
    out = pl.pallas_call(
        _mlp_kernel,
        out_shape=jax.ShapeDtypeStruct((_C, n_pad), jnp.float32),
        grid=(grid,),
        in_specs=[
            pl.BlockSpec((9, 4, _TN), lambda i: (0, 0, i)),
            pl.BlockSpec((_H, _PATCH), lambda i: (0, 0)),
            pl.BlockSpec((_H, 1), lambda i: (0, 0)),
            pl.BlockSpec((_C, _H), lambda i: (0, 0)),
            pl.BlockSpec((_C, 1), lambda i: (0, 0)),
        ],
        out_specs=pl.BlockSpec((_C, _TN), lambda i: (0, i)),
        compiler_params=pltpu.CompilerParams(dimension_semantics=("parallel",)),
    )(xt, w1t, b1, w2t, b2)
    return jnp.transpose(out[:, :n])
